# baseline (speedup 1.0000x reference)
import functools

import jax
import jax.numpy as jnp
import numpy as np
from jax.experimental import pallas as pl
from jax.experimental.pallas import tpu as pltpu

N0, N1, N2 = 50000, 10000, 2048
IN_DIM, HID, HEADS, OUT_DIM = 128, 128, 4, 128
E1, E2 = 160000, 32768


def _matmul_kernel(x_ref, w_ref, b_ref, o_ref):
    o_ref[...] = (
        jnp.dot(x_ref[...], w_ref[...], preferred_element_type=jnp.float32)
        + b_ref[...]
    )


def _matmul(x, w, b, block_m=1024):
    m, kdim = x.shape
    n = w.shape[1]
    pad_m = (-m) % block_m
    if pad_m:
        x = jnp.pad(x, ((0, pad_m), (0, 0)))
    mp = x.shape[0]
    out = pl.pallas_call(
        _matmul_kernel,
        grid=(mp // block_m,),
        in_specs=[
            pl.BlockSpec((block_m, kdim), lambda i: (i, 0)),
            pl.BlockSpec((kdim, n), lambda i: (0, 0)),
            pl.BlockSpec((n,), lambda i: (0,)),
        ],
        out_specs=pl.BlockSpec((block_m, n), lambda i: (i, 0)),
        out_shape=jax.ShapeDtypeStruct((mp, n), jnp.float32),
    )(x, w, b)
    return out[:m]


def _transformer_conv(x_src, x_dst, src, dst, Wq, bq, Wk, bk, Wv, bv,
                      Wskip, bskip, Wbeta, H, C, n_dst):
    q = _matmul(x_dst, Wq, bq).reshape(-1, H, C)
    k = _matmul(x_src, Wk, bk).reshape(-1, H, C)
    v = _matmul(x_src, Wv, bv).reshape(-1, H, C)
    score = jnp.sum(q[dst] * k[src], axis=-1) / np.sqrt(C)
    m = jax.ops.segment_max(score, dst, num_segments=n_dst)
    m = jnp.where(jnp.isfinite(m), m, 0.0)
    e = jnp.exp(score - m[dst])
    denom = jax.ops.segment_sum(e, dst, num_segments=n_dst)
    alpha = e / (denom[dst] + 1e-16)
    out = jax.ops.segment_sum(v[src] * alpha[:, :, None], dst, num_segments=n_dst)
    out = out.reshape(n_dst, H * C)
    r = _matmul(x_dst, Wskip, bskip)
    g = jax.nn.sigmoid(jnp.concatenate([out, r, out - r], axis=-1) @ Wbeta)
    return g * r + (1.0 - g) * out


def kernel(x, Wq1, bq1, Wk1, bk1, Wv1, bv1, Wskip1, bskip1, Wbeta1,
           bn_gamma, bn_beta, bn_mean, bn_var, Wq2, bq2, Wk2, bk2, Wv2, bv2,
           Wskip2, bskip2, Wbeta2, edge_src1, edge_dst1, edge_src2, edge_dst2):
    h = _transformer_conv(x, x[:N1], edge_src1, edge_dst1, Wq1, bq1, Wk1, bk1,
                          Wv1, bv1, Wskip1, bskip1, Wbeta1, HEADS, HID, N1)
    h = (h - bn_mean) / jnp.sqrt(bn_var + 1e-5) * bn_gamma + bn_beta
    h = jax.nn.elu(h)
    out = _transformer_conv(h, h[:N2], edge_src2, edge_dst2, Wq2, bq2, Wk2, bk2,
                            Wv2, bv2, Wskip2, bskip2, Wbeta2, 1, OUT_DIM, N2)
    return out
